# baseline (speedup 1.0000x reference)
import jax
import jax.numpy as jnp
from jax import lax
from jax.experimental import pallas as pl
from jax.experimental.pallas import tpu as pltpu
from jax.experimental.pallas import tpu_sc as plsc

E = 8
K = 2
H = 1024
F = 768
T = 2048
A = T * K
TM = 256
G = 24
P_MAX = G * TM

NW = 32
APT = A // NW
TMC = 256


def _router_body(x_ref, wg_ref, topi_ref, topv_ref, pos_ref, te_ref, tc_ref,
                 xb_ref):
    x = x_ref[...]
    wg = wg_ref[...]
    logits = lax.dot_general(x, wg, (((1,), (1,)), ((), ())),
                             preferred_element_type=jnp.float32)
    m = jnp.max(logits, axis=-1, keepdims=True)
    ex = jnp.exp(logits - m)
    probs = ex / jnp.sum(ex, axis=-1, keepdims=True)
    lane = lax.broadcasted_iota(jnp.int32, probs.shape, 1)
    v1 = jnp.max(probs, axis=-1, keepdims=True)
    i1 = jnp.argmax(probs, axis=-1).astype(jnp.int32)[:, None]
    masked = jnp.where(lane == i1, -jnp.inf, probs)
    v2 = jnp.max(masked, axis=-1, keepdims=True)
    i2 = jnp.argmax(masked, axis=-1).astype(jnp.int32)[:, None]
    s = v1 + v2
    topi_ref[...] = jnp.concatenate([i1, i2], axis=1)
    topv_ref[...] = jnp.concatenate([v1 / s, v2 / s], axis=1)
    xb_ref[...] = x.astype(jnp.bfloat16)

    oh = jnp.concatenate([(lane == i1).astype(jnp.float32),
                          (lane == i2).astype(jnp.float32)], axis=0)
    cs = oh
    k = 1
    while k < A:
        cs = cs + jnp.concatenate(
            [jnp.zeros((k, E), jnp.float32), cs[:A - k]], axis=0)
        k *= 2
    counts = cs[A - 1:A]
    padded = jnp.ceil(counts / TM) * TM
    lower8 = (lax.broadcasted_iota(jnp.int32, (E, E), 0)
              <= lax.broadcasted_iota(jnp.int32, (E, E), 1)).astype(jnp.float32)
    incl = jnp.dot(padded, lower8, preferred_element_type=jnp.float32)
    poff = incl - padded
    pos = jnp.sum(oh * (cs - 1.0 + poff), axis=1, keepdims=True)
    pos_ref[...] = pos.astype(jnp.int32)

    ts = lax.broadcasted_iota(jnp.int32, (NW, E), 0).astype(jnp.float32) * TM
    acc = jnp.sum((incl <= ts).astype(jnp.float32), axis=1, keepdims=True)
    te = jnp.clip(acc, 0.0, E - 1)
    teoh = (lax.broadcasted_iota(jnp.int32, (NW, E), 1).astype(jnp.float32)
            == te).astype(jnp.float32)
    cnt_te = jnp.sum(teoh * counts, axis=1, keepdims=True)
    poff_te = jnp.sum(teoh * poff, axis=1, keepdims=True)
    tcv = jnp.clip(cnt_te - (ts[:, :1] - poff_te), 0.0, TM)
    te_ref[...] = te.reshape(1, NW).astype(jnp.int32)
    tc_ref[...] = tcv.reshape(1, NW).astype(jnp.int32)


def _route(x, wg):
    return pl.pallas_call(
        _router_body,
        out_shape=(
            jax.ShapeDtypeStruct((T, K), jnp.int32),
            jax.ShapeDtypeStruct((T, K), jnp.float32),
            jax.ShapeDtypeStruct((A, 1), jnp.int32),
            jax.ShapeDtypeStruct((1, NW), jnp.int32),
            jax.ShapeDtypeStruct((1, NW), jnp.int32),
            jax.ShapeDtypeStruct((T, H), jnp.bfloat16),
        ),
    )(x, wg)


_NB = 4


def _dispatch_body(pos_hbm, x_hbm, xs_hbm, posb_v,
                   p0, p1, p2, p3, b0, b1, b2, b3,
                   sg0, sg1, sg2, sg3, ss0, ss1, ss2, ss3):
    wid = lax.axis_index("s") * 2 + lax.axis_index("c")
    base_a = wid * APT
    pltpu.sync_copy(pos_hbm.at[pl.ds(base_a, APT)], posb_v)
    iota = lax.iota(jnp.int32, 16)
    pr = [p0, p1, p2, p3]
    br = [b0, b1, b2, b3]
    sg = [sg0, sg1, sg2, sg3]
    ss = [ss0, ss1, ss2, ss3]
    nv = APT // 16

    def gath(v):
        tvec = (jnp.full((16,), base_a + v * 16, jnp.int32) + iota) & (T - 1)
        return pltpu.async_copy(x_hbm.at[tvec], br[v % _NB], sg[v % _NB])

    gd = {v: gath(v) for v in range(_NB)}
    sd = {}
    for v in range(nv):
        b = v % _NB
        if v >= _NB:
            sd[v - _NB].wait()
            gd[v] = gath(v)
        gd[v].wait()
        pr[b][...] = posb_v[pl.ds(v * 16, 16)]
        sd[v] = pltpu.async_copy(br[b], xs_hbm.at[pr[b]], ss[b])
    for v in range(nv - _NB, nv):
        sd[v].wait()


def _dispatch(pos_flat, x):
    mesh = plsc.VectorSubcoreMesh(core_axis_name="c", subcore_axis_name="s")
    return pl.kernel(
        _dispatch_body,
        out_type=jax.ShapeDtypeStruct((P_MAX, H // 2), jnp.int32),
        mesh=mesh,
        scratch_types=(
            [pltpu.VMEM((APT,), jnp.int32)]
            + [pltpu.VMEM((16,), jnp.int32)] * _NB
            + [pltpu.VMEM((16, H // 2), jnp.int32)] * _NB
            + [pltpu.SemaphoreType.DMA] * (2 * _NB)
        ),
    )(pos_flat, x)


def _gmm_body(te_ref, tc_ref, x_ref, wgu_ref, wd_ref, y_ref):
    g = pl.program_id(0)
    cnt = tc_ref[g]

    @pl.when(cnt > 0)
    def _():
        x = x_ref[...]
        h = jnp.dot(x, wgu_ref[0].astype(jnp.bfloat16),
                    preferred_element_type=jnp.float32)
        gate = h[:, :F]
        up = h[:, F:]
        act = gate * jax.nn.sigmoid(gate) * up
        y_ref[...] = jnp.dot(act.astype(jnp.bfloat16),
                             wd_ref[0].astype(jnp.bfloat16),
                             preferred_element_type=jnp.float32
                             ).astype(jnp.bfloat16)


def _gmm(te, tcnt, x_sorted, w_gate_up, w_down):
    grid_spec = pltpu.PrefetchScalarGridSpec(
        num_scalar_prefetch=2,
        grid=(G,),
        in_specs=[
            pl.BlockSpec((TM, H), lambda g, te, tc: (g, 0)),
            pl.BlockSpec((1, H, 2 * F), lambda g, te, tc: (te[g], 0, 0)),
            pl.BlockSpec((1, F, H), lambda g, te, tc: (te[g], 0, 0)),
        ],
        out_specs=pl.BlockSpec((TM, H), lambda g, te, tc: (g, 0)),
    )
    return pl.pallas_call(
        _gmm_body,
        grid_spec=grid_spec,
        out_shape=jax.ShapeDtypeStruct((P_MAX, H), jnp.bfloat16),
    )(te, tcnt, x_sorted, w_gate_up, w_down)


def _cgather_body(y_hbm, pos_hbm, yg_hbm, posb_v,
                  b0, b1, b2, b3, sg0, sg1, sg2, sg3, ss0, ss1, ss2, ss3):
    wid = lax.axis_index("s") * 2 + lax.axis_index("c")
    base_a = wid * APT
    pltpu.sync_copy(pos_hbm.at[pl.ds(base_a, APT)], posb_v)
    br = [b0, b1, b2, b3]
    sg = [sg0, sg1, sg2, sg3]
    ss = [ss0, ss1, ss2, ss3]
    nv = APT // 16

    def gath(v):
        pidx = posb_v[pl.ds(v * 16, 16)]
        return pltpu.async_copy(y_hbm.at[pidx], br[v % _NB], sg[v % _NB])

    gd = {v: gath(v) for v in range(_NB)}
    sd = {}
    for v in range(nv):
        b = v % _NB
        if v >= _NB:
            sd[v - _NB].wait()
            gd[v] = gath(v)
        gd[v].wait()
        sd[v] = pltpu.async_copy(
            br[b], yg_hbm.at[pl.ds(base_a + v * 16, 16)], ss[b])
    for v in range(nv - _NB, nv):
        sd[v].wait()


def _cgather(y_sorted, pos_flat):
    mesh = plsc.VectorSubcoreMesh(core_axis_name="c", subcore_axis_name="s")
    return pl.kernel(
        _cgather_body,
        out_type=jax.ShapeDtypeStruct((A, H // 2), jnp.int32),
        mesh=mesh,
        scratch_types=(
            [pltpu.VMEM((APT,), jnp.int32)]
            + [pltpu.VMEM((16, H // 2), jnp.int32)] * _NB
            + [pltpu.SemaphoreType.DMA] * (2 * _NB)
        ),
    )(y_sorted, pos_flat)


def _wsum_body(w_ref, yg_ref, o_ref):
    w = w_ref[...]
    y3 = yg_ref[...].astype(jnp.float32)
    o_ref[...] = w[:, :1] * y3[0] + w[:, 1:2] * y3[1]


def _wsum(topv, yg3):
    return pl.pallas_call(
        _wsum_body,
        grid=(T // TMC,),
        in_specs=[
            pl.BlockSpec((TMC, K), lambda i: (i, 0)),
            pl.BlockSpec((K, TMC, H), lambda i: (0, i, 0)),
        ],
        out_specs=pl.BlockSpec((TMC, H), lambda i: (i, 0)),
        out_shape=jax.ShapeDtypeStruct((T, H), jnp.float32),
    )(topv, yg3)


def _to_i32(a):
    return lax.bitcast_convert_type(a.reshape(a.shape[0], H // 2, 2), jnp.int32)


def _to_bf16(a):
    return lax.bitcast_convert_type(a, jnp.bfloat16).reshape(a.shape[0], H)


def kernel(hidden_states, Wg, W_gate_up, W_down):
    x = hidden_states
    topi, topv, pos2d, te2d, tc2d, xb = _route(x, Wg)
    pos_flat = pos2d.reshape(A)
    xs32 = _dispatch(pos_flat, _to_i32(xb))
    te = te2d.reshape(NW)[:G]
    tcnt = tc2d.reshape(NW)[:G]
    y_sorted = _gmm(te, tcnt, _to_bf16(xs32), W_gate_up, W_down)
    yg32 = _cgather(_to_i32(y_sorted), pos_flat)
    return _wsum(topv, _to_bf16(yg32).reshape(K, T, H))


# speedup vs baseline: 4.0985x; 4.0985x over previous
import jax
import jax.numpy as jnp
from jax import lax
from jax.experimental import pallas as pl
from jax.experimental.pallas import tpu as pltpu
from jax.experimental.pallas import tpu_sc as plsc

E = 8
K = 2
H = 1024
F = 768
T = 2048
A = T * K
TM = 256
G = 24
P_MAX = G * TM

NW = 32
APT = A // NW
TMC = 256


def _router_body(x_ref, wg_ref, topi_ref, topv_ref, pos_ref, te_ref, tc_ref):
    x = x_ref[...]
    wg = wg_ref[...]
    logits = lax.dot_general(x, wg, (((1,), (1,)), ((), ())),
                             preferred_element_type=jnp.float32)
    m = jnp.max(logits, axis=-1, keepdims=True)
    ex = jnp.exp(logits - m)
    probs = ex / jnp.sum(ex, axis=-1, keepdims=True)
    lane = lax.broadcasted_iota(jnp.int32, probs.shape, 1)
    v1 = jnp.max(probs, axis=-1, keepdims=True)
    i1 = jnp.argmax(probs, axis=-1).astype(jnp.int32)[:, None]
    masked = jnp.where(lane == i1, -jnp.inf, probs)
    v2 = jnp.max(masked, axis=-1, keepdims=True)
    i2 = jnp.argmax(masked, axis=-1).astype(jnp.int32)[:, None]
    s = v1 + v2
    topi_ref[...] = jnp.concatenate([i1, i2], axis=1)
    topv_ref[...] = jnp.concatenate([v1 / s, v2 / s], axis=1)

    oh = jnp.concatenate([(lane == i1).astype(jnp.float32),
                          (lane == i2).astype(jnp.float32)], axis=0)
    cs = oh
    k = 1
    while k < A:
        cs = cs + jnp.concatenate(
            [jnp.zeros((k, E), jnp.float32), cs[:A - k]], axis=0)
        k *= 2
    counts = cs[A - 1:A]
    padded = jnp.ceil(counts / TM) * TM
    lower8 = (lax.broadcasted_iota(jnp.int32, (E, E), 0)
              <= lax.broadcasted_iota(jnp.int32, (E, E), 1)).astype(jnp.float32)
    incl = jnp.dot(padded, lower8, preferred_element_type=jnp.float32)
    poff = incl - padded
    pos = jnp.sum(oh * (cs - 1.0 + poff), axis=1, keepdims=True)
    pos_ref[...] = pos.astype(jnp.int32)

    ts = lax.broadcasted_iota(jnp.int32, (NW, E), 0).astype(jnp.float32) * TM
    acc = jnp.sum((incl <= ts).astype(jnp.float32), axis=1, keepdims=True)
    te = jnp.clip(acc, 0.0, E - 1)
    teoh = (lax.broadcasted_iota(jnp.int32, (NW, E), 1).astype(jnp.float32)
            == te).astype(jnp.float32)
    cnt_te = jnp.sum(teoh * counts, axis=1, keepdims=True)
    poff_te = jnp.sum(teoh * poff, axis=1, keepdims=True)
    tcv = jnp.clip(cnt_te - (ts[:, :1] - poff_te), 0.0, TM)
    te_ref[...] = te.reshape(1, NW).astype(jnp.int32)
    tc_ref[...] = tcv.reshape(1, NW).astype(jnp.int32)


def _route(x, wg):
    return pl.pallas_call(
        _router_body,
        out_shape=(
            jax.ShapeDtypeStruct((T, K), jnp.int32),
            jax.ShapeDtypeStruct((T, K), jnp.float32),
            jax.ShapeDtypeStruct((A, 1), jnp.int32),
            jax.ShapeDtypeStruct((1, NW), jnp.int32),
            jax.ShapeDtypeStruct((1, NW), jnp.int32),
        ),
    )(x, wg)


_NB = 4


def _dispatch_body(pos_hbm, x_hbm, xs_hbm, posb_v,
                   p0, p1, p2, p3, b0, b1, b2, b3,
                   sg0, sg1, sg2, sg3, ss0, ss1, ss2, ss3):
    wid = lax.axis_index("s") * 2 + lax.axis_index("c")
    base_a = wid * APT
    pltpu.sync_copy(pos_hbm.at[pl.ds(base_a, APT)], posb_v)
    iota = lax.iota(jnp.int32, 16)
    pr = [p0, p1, p2, p3]
    br = [b0, b1, b2, b3]
    sg = [sg0, sg1, sg2, sg3]
    ss = [ss0, ss1, ss2, ss3]
    nv = APT // 16

    def gath(v):
        tvec = (jnp.full((16,), base_a + v * 16, jnp.int32) + iota) & (T - 1)
        return pltpu.async_copy(x_hbm.at[tvec], br[v % _NB], sg[v % _NB])

    gd = {v: gath(v) for v in range(_NB)}
    sd = {}
    for v in range(nv):
        b = v % _NB
        if v >= _NB:
            sd[v - _NB].wait()
            gd[v] = gath(v)
        gd[v].wait()
        pr[b][...] = posb_v[pl.ds(v * 16, 16)]
        sd[v] = pltpu.async_copy(br[b], xs_hbm.at[pr[b]], ss[b])
    for v in range(nv - _NB, nv):
        sd[v].wait()


def _dispatch(pos_flat, x):
    mesh = plsc.VectorSubcoreMesh(core_axis_name="c", subcore_axis_name="s")
    return pl.kernel(
        _dispatch_body,
        out_type=jax.ShapeDtypeStruct((P_MAX, H), jnp.float32),
        mesh=mesh,
        scratch_types=(
            [pltpu.VMEM((APT,), jnp.int32)]
            + [pltpu.VMEM((16,), jnp.int32)] * _NB
            + [pltpu.VMEM((16, H), jnp.float32)] * _NB
            + [pltpu.SemaphoreType.DMA] * (2 * _NB)
        ),
    )(pos_flat, x)


def _gmm_body(te_ref, tc_ref, x_ref, wgu_ref, wd_ref, y_ref):
    g = pl.program_id(0)
    cnt = tc_ref[g]

    @pl.when(cnt > 0)
    def _():
        x = x_ref[...].astype(jnp.bfloat16)
        h = jnp.dot(x, wgu_ref[0].astype(jnp.bfloat16),
                    preferred_element_type=jnp.float32)
        gate = h[:, :F]
        up = h[:, F:]
        act = gate * jax.nn.sigmoid(gate) * up
        y_ref[...] = jnp.dot(act.astype(jnp.bfloat16),
                             wd_ref[0].astype(jnp.bfloat16),
                             preferred_element_type=jnp.float32)


def _gmm(te, tcnt, x_sorted, w_gate_up, w_down):
    grid_spec = pltpu.PrefetchScalarGridSpec(
        num_scalar_prefetch=2,
        grid=(G,),
        in_specs=[
            pl.BlockSpec((TM, H), lambda g, te, tc: (g, 0)),
            pl.BlockSpec((1, H, 2 * F), lambda g, te, tc: (te[g], 0, 0)),
            pl.BlockSpec((1, F, H), lambda g, te, tc: (te[g], 0, 0)),
        ],
        out_specs=pl.BlockSpec((TM, H), lambda g, te, tc: (g, 0)),
    )
    return pl.pallas_call(
        _gmm_body,
        grid_spec=grid_spec,
        out_shape=jax.ShapeDtypeStruct((P_MAX, H), jnp.float32),
    )(te, tcnt, x_sorted, w_gate_up, w_down)


def _cgather_body(y_hbm, pos_hbm, yg_hbm, posb_v,
                  b0, b1, b2, b3, sg0, sg1, sg2, sg3, ss0, ss1, ss2, ss3):
    wid = lax.axis_index("s") * 2 + lax.axis_index("c")
    base_a = wid * APT
    pltpu.sync_copy(pos_hbm.at[pl.ds(base_a, APT)], posb_v)
    br = [b0, b1, b2, b3]
    sg = [sg0, sg1, sg2, sg3]
    ss = [ss0, ss1, ss2, ss3]
    nv = APT // 16

    def gath(v):
        pidx = posb_v[pl.ds(v * 16, 16)]
        return pltpu.async_copy(y_hbm.at[pidx], br[v % _NB], sg[v % _NB])

    gd = {v: gath(v) for v in range(_NB)}
    sd = {}
    for v in range(nv):
        b = v % _NB
        if v >= _NB:
            sd[v - _NB].wait()
            gd[v] = gath(v)
        gd[v].wait()
        sd[v] = pltpu.async_copy(
            br[b], yg_hbm.at[pl.ds(base_a + v * 16, 16)], ss[b])
    for v in range(nv - _NB, nv):
        sd[v].wait()


def _cgather(y_sorted, pos_flat):
    mesh = plsc.VectorSubcoreMesh(core_axis_name="c", subcore_axis_name="s")
    return pl.kernel(
        _cgather_body,
        out_type=jax.ShapeDtypeStruct((A, H), jnp.float32),
        mesh=mesh,
        scratch_types=(
            [pltpu.VMEM((APT,), jnp.int32)]
            + [pltpu.VMEM((16, H), jnp.float32)] * _NB
            + [pltpu.SemaphoreType.DMA] * (2 * _NB)
        ),
    )(y_sorted, pos_flat)


def _wsum_body(w_ref, yg_ref, o_ref):
    w = w_ref[...]
    y3 = yg_ref[...]
    o_ref[...] = w[:, :1] * y3[0] + w[:, 1:2] * y3[1]


def _wsum(topv, yg3):
    return pl.pallas_call(
        _wsum_body,
        grid=(T // TMC,),
        in_specs=[
            pl.BlockSpec((TMC, K), lambda i: (i, 0)),
            pl.BlockSpec((K, TMC, H), lambda i: (0, i, 0)),
        ],
        out_specs=pl.BlockSpec((TMC, H), lambda i: (i, 0)),
        out_shape=jax.ShapeDtypeStruct((T, H), jnp.float32),
    )(topv, yg3)


def kernel(hidden_states, Wg, W_gate_up, W_down):
    x = hidden_states
    topi, topv, pos2d, te2d, tc2d = _route(x, Wg)
    pos_flat = pos2d.reshape(A)
    x_sorted = _dispatch(pos_flat, x)
    te = te2d.reshape(NW)[:G]
    tcnt = tc2d.reshape(NW)[:G]
    y_sorted = _gmm(te, tcnt, x_sorted, W_gate_up, W_down)
    yg = _cgather(y_sorted, pos_flat)
    return _wsum(topv, yg.reshape(K, T, H))


# speedup vs baseline: 4.4238x; 1.0794x over previous
import jax
import jax.numpy as jnp
from jax import lax
from jax.experimental import pallas as pl
from jax.experimental.pallas import tpu as pltpu
from jax.experimental.pallas import tpu_sc as plsc

E = 8
K = 2
H = 1024
F = 768
T = 2048
A = T * K
TM = 256
G = 24
P_MAX = G * TM

NW = 32
APT = A // NW
TMC = 256


def _router_body(x_ref, wg_ref, pos_ref, te_ref, tc_ref, wrep_ref):
    x = x_ref[...]
    wg = wg_ref[...]
    logits = lax.dot_general(x, wg, (((1,), (1,)), ((), ())),
                             preferred_element_type=jnp.float32)
    m = jnp.max(logits, axis=-1, keepdims=True)
    ex = jnp.exp(logits - m)
    probs = ex / jnp.sum(ex, axis=-1, keepdims=True)
    lane = lax.broadcasted_iota(jnp.int32, probs.shape, 1)
    v1 = jnp.max(probs, axis=-1, keepdims=True)
    i1 = jnp.argmax(probs, axis=-1).astype(jnp.int32)[:, None]
    masked = jnp.where(lane == i1, -jnp.inf, probs)
    v2 = jnp.max(masked, axis=-1, keepdims=True)
    i2 = jnp.argmax(masked, axis=-1).astype(jnp.int32)[:, None]
    s = v1 + v2
    wcol = jnp.concatenate([v1 / s, v2 / s], axis=0)
    wrep_ref[...] = jnp.broadcast_to(wcol, (A, 16))

    oh = jnp.concatenate([(lane == i1).astype(jnp.float32),
                          (lane == i2).astype(jnp.float32)], axis=0)
    cs = oh
    k = 1
    while k < A:
        cs = cs + jnp.concatenate(
            [jnp.zeros((k, E), jnp.float32), cs[:A - k]], axis=0)
        k *= 2
    counts = cs[A - 1:A]
    padded = jnp.ceil(counts / TM) * TM
    lower8 = (lax.broadcasted_iota(jnp.int32, (E, E), 0)
              <= lax.broadcasted_iota(jnp.int32, (E, E), 1)).astype(jnp.float32)
    incl = jnp.dot(padded, lower8, preferred_element_type=jnp.float32)
    poff = incl - padded
    pos = jnp.sum(oh * (cs - 1.0 + poff), axis=1, keepdims=True)
    pos_ref[...] = pos.astype(jnp.int32)

    ts = lax.broadcasted_iota(jnp.int32, (NW, E), 0).astype(jnp.float32) * TM
    acc = jnp.sum((incl <= ts).astype(jnp.float32), axis=1, keepdims=True)
    te = jnp.clip(acc, 0.0, E - 1)
    teoh = (lax.broadcasted_iota(jnp.int32, (NW, E), 1).astype(jnp.float32)
            == te).astype(jnp.float32)
    cnt_te = jnp.sum(teoh * counts, axis=1, keepdims=True)
    poff_te = jnp.sum(teoh * poff, axis=1, keepdims=True)
    tcv = jnp.clip(cnt_te - (ts[:, :1] - poff_te), 0.0, TM)
    te_ref[...] = te.reshape(1, NW).astype(jnp.int32)
    tc_ref[...] = tcv.reshape(1, NW).astype(jnp.int32)


def _route(x, wg):
    return pl.pallas_call(
        _router_body,
        out_shape=(
            jax.ShapeDtypeStruct((A, 1), jnp.int32),
            jax.ShapeDtypeStruct((1, NW), jnp.int32),
            jax.ShapeDtypeStruct((1, NW), jnp.int32),
            jax.ShapeDtypeStruct((A, 16), jnp.float32),
        ),
    )(x, wg)


_NB = 4


def _dispatch_body(pos_hbm, x_hbm, xs_hbm, posb_v,
                   p0, p1, p2, p3, b0, b1, b2, b3,
                   sg0, sg1, sg2, sg3, ss0, ss1, ss2, ss3):
    wid = lax.axis_index("s") * 2 + lax.axis_index("c")
    base_a = wid * APT
    pltpu.sync_copy(pos_hbm.at[pl.ds(base_a, APT)], posb_v)
    iota = lax.iota(jnp.int32, 16)
    pr = [p0, p1, p2, p3]
    br = [b0, b1, b2, b3]
    sg = [sg0, sg1, sg2, sg3]
    ss = [ss0, ss1, ss2, ss3]
    nv = APT // 16

    def gath(v):
        tvec = (jnp.full((16,), base_a + v * 16, jnp.int32) + iota) & (T - 1)
        return pltpu.async_copy(x_hbm.at[tvec], br[v % _NB], sg[v % _NB])

    gd = {v: gath(v) for v in range(_NB)}
    sd = {}
    for v in range(nv):
        b = v % _NB
        if v >= _NB:
            sd[v - _NB].wait()
            gd[v] = gath(v)
        gd[v].wait()
        pr[b][...] = posb_v[pl.ds(v * 16, 16)]
        sd[v] = pltpu.async_copy(br[b], xs_hbm.at[pr[b]], ss[b])
    for v in range(nv - _NB, nv):
        sd[v].wait()


def _dispatch(pos_flat, x):
    mesh = plsc.VectorSubcoreMesh(core_axis_name="c", subcore_axis_name="s")
    return pl.kernel(
        _dispatch_body,
        out_type=jax.ShapeDtypeStruct((P_MAX, H), jnp.float32),
        mesh=mesh,
        scratch_types=(
            [pltpu.VMEM((APT,), jnp.int32)]
            + [pltpu.VMEM((16,), jnp.int32)] * _NB
            + [pltpu.VMEM((16, H), jnp.float32)] * _NB
            + [pltpu.SemaphoreType.DMA] * (2 * _NB)
        ),
    )(pos_flat, x)


def _gmm_body(te_ref, tc_ref, x_ref, wgu_ref, wd_ref, y_ref):
    g = pl.program_id(0)
    cnt = tc_ref[g]

    @pl.when(cnt > 0)
    def _():
        x = x_ref[...].astype(jnp.bfloat16)
        h = jnp.dot(x, wgu_ref[0].astype(jnp.bfloat16),
                    preferred_element_type=jnp.float32)
        gate = h[:, :F]
        up = h[:, F:]
        act = gate * jax.nn.sigmoid(gate) * up
        y_ref[...] = jnp.dot(act.astype(jnp.bfloat16),
                             wd_ref[0].astype(jnp.bfloat16),
                             preferred_element_type=jnp.float32)


def _gmm(te, tcnt, x_sorted, w_gate_up, w_down):
    grid_spec = pltpu.PrefetchScalarGridSpec(
        num_scalar_prefetch=2,
        grid=(G,),
        in_specs=[
            pl.BlockSpec((TM, H), lambda g, te, tc: (g, 0)),
            pl.BlockSpec((1, H, 2 * F), lambda g, te, tc: (te[g], 0, 0)),
            pl.BlockSpec((1, F, H), lambda g, te, tc: (te[g], 0, 0)),
        ],
        out_specs=pl.BlockSpec((TM, H), lambda g, te, tc: (g, 0)),
    )
    return pl.pallas_call(
        _gmm_body,
        grid_spec=grid_spec,
        out_shape=jax.ShapeDtypeStruct((P_MAX, H), jnp.float32),
    )(te, tcnt, x_sorted, w_gate_up, w_down)


TPT = T // NW


def _combine_body(y_hbm, pos_hbm, wrep_hbm, out_hbm,
                  pb0_v, pb1_v, wb0_v, wb1_v,
                  ya0, yb0, ya1, yb1, ob0, ob1,
                  sa0, sb0, sa1, sb1, so0, so1):
    wid = lax.axis_index("s") * 2 + lax.axis_index("c")
    base_t = wid * TPT
    pltpu.sync_copy(pos_hbm.at[pl.ds(base_t, TPT)], pb0_v)
    pltpu.sync_copy(pos_hbm.at[pl.ds(T + base_t, TPT)], pb1_v)
    pltpu.sync_copy(wrep_hbm.at[pl.ds(base_t, TPT)], wb0_v)
    pltpu.sync_copy(wrep_hbm.at[pl.ds(T + base_t, TPT)], wb1_v)
    ya = [ya0, ya1]
    yb = [yb0, yb1]
    ob = [ob0, ob1]
    sa = [sa0, sa1]
    sb = [sb0, sb1]
    so = [so0, so1]
    nq = TPT // 16

    def gath(q):
        s = q % 2
        p0 = pb0_v[pl.ds(q * 16, 16)]
        p1 = pb1_v[pl.ds(q * 16, 16)]
        return (pltpu.async_copy(y_hbm.at[p0], ya[s], sa[s]),
                pltpu.async_copy(y_hbm.at[p1], yb[s], sb[s]))

    gd = {0: gath(0)}
    sd = {}
    for q in range(nq):
        s = q % 2
        gd[q][0].wait()
        gd[q][1].wait()
        if q + 1 < nq:
            gd[q + 1] = gath(q + 1)
        if q >= 2:
            sd[q - 2].wait()

        def row_body(i, carry, _q=q, _s=s):
            w0 = wb0_v[_q * 16 + i]
            w1 = wb1_v[_q * 16 + i]
            for c in range(H // 16):
                off = c * 16
                ob[_s][i, pl.ds(off, 16)] = (
                    w0 * ya[_s][i, pl.ds(off, 16)]
                    + w1 * yb[_s][i, pl.ds(off, 16)])
            return carry

        lax.fori_loop(0, 16, row_body, 0)
        sd[q] = pltpu.async_copy(
            ob[s], out_hbm.at[pl.ds(base_t + q * 16, 16)], so[s])
    sd[nq - 2].wait()
    sd[nq - 1].wait()


def _combine(y_sorted, pos_flat, wrep):
    mesh = plsc.VectorSubcoreMesh(core_axis_name="c", subcore_axis_name="s")
    return pl.kernel(
        _combine_body,
        out_type=jax.ShapeDtypeStruct((T, H), jnp.float32),
        mesh=mesh,
        scratch_types=(
            [pltpu.VMEM((TPT,), jnp.int32)] * 2
            + [pltpu.VMEM((TPT, 16), jnp.float32)] * 2
            + [pltpu.VMEM((16, H), jnp.float32)] * 6
            + [pltpu.SemaphoreType.DMA] * 6
        ),
    )(y_sorted, pos_flat, wrep)


def kernel(hidden_states, Wg, W_gate_up, W_down):
    x = hidden_states
    pos2d, te2d, tc2d, wrep = _route(x, Wg)
    pos_flat = pos2d.reshape(A)
    x_sorted = _dispatch(pos_flat, x)
    te = te2d.reshape(NW)[:G]
    tcnt = tc2d.reshape(NW)[:G]
    y_sorted = _gmm(te, tcnt, x_sorted, W_gate_up, W_down)
    return _combine(y_sorted, pos_flat, wrep)


# speedup vs baseline: 4.6449x; 1.0500x over previous
import jax
import jax.numpy as jnp
from jax import lax
from jax.experimental import pallas as pl
from jax.experimental.pallas import tpu as pltpu
from jax.experimental.pallas import tpu_sc as plsc

E = 8
K = 2
H = 1024
F = 768
T = 2048
A = T * K
TM = 256
G = 24
P_MAX = G * TM

NW = 32
APT = A // NW
TMC = 256


def _router_body(x_ref, wg_ref, pos_ref, te_ref, tc_ref, xb_ref, wrep_ref):
    x = x_ref[...]
    wg = wg_ref[...]
    logits = lax.dot_general(x, wg, (((1,), (1,)), ((), ())),
                             preferred_element_type=jnp.float32)
    m = jnp.max(logits, axis=-1, keepdims=True)
    ex = jnp.exp(logits - m)
    probs = ex / jnp.sum(ex, axis=-1, keepdims=True)
    lane = lax.broadcasted_iota(jnp.int32, probs.shape, 1)
    v1 = jnp.max(probs, axis=-1, keepdims=True)
    i1 = jnp.argmax(probs, axis=-1).astype(jnp.int32)[:, None]
    masked = jnp.where(lane == i1, -jnp.inf, probs)
    v2 = jnp.max(masked, axis=-1, keepdims=True)
    i2 = jnp.argmax(masked, axis=-1).astype(jnp.int32)[:, None]
    s = v1 + v2
    wcol = jnp.concatenate([v1 / s, v2 / s], axis=0)
    wrep_ref[...] = jnp.broadcast_to(wcol, (A, 16))

    oh = jnp.concatenate([(lane == i1).astype(jnp.float32),
                          (lane == i2).astype(jnp.float32)], axis=0)
    cs = oh
    k = 1
    while k < A:
        cs = cs + jnp.concatenate(
            [jnp.zeros((k, E), jnp.float32), cs[:A - k]], axis=0)
        k *= 2
    counts = cs[A - 1:A]
    padded = jnp.ceil(counts / TM) * TM
    lower8 = (lax.broadcasted_iota(jnp.int32, (E, E), 0)
              <= lax.broadcasted_iota(jnp.int32, (E, E), 1)).astype(jnp.float32)
    incl = jnp.dot(padded, lower8, preferred_element_type=jnp.float32)
    poff = incl - padded
    pos = jnp.sum(oh * (cs - 1.0 + poff), axis=1, keepdims=True)
    pos_ref[...] = pos.astype(jnp.int32)

    ts = lax.broadcasted_iota(jnp.int32, (NW, E), 0).astype(jnp.float32) * TM
    acc = jnp.sum((incl <= ts).astype(jnp.float32), axis=1, keepdims=True)
    te = jnp.clip(acc, 0.0, E - 1)
    teoh = (lax.broadcasted_iota(jnp.int32, (NW, E), 1).astype(jnp.float32)
            == te).astype(jnp.float32)
    cnt_te = jnp.sum(teoh * counts, axis=1, keepdims=True)
    poff_te = jnp.sum(teoh * poff, axis=1, keepdims=True)
    tcv = jnp.clip(cnt_te - (ts[:, :1] - poff_te), 0.0, TM)
    active = (tcv > 0.0).astype(jnp.float32)
    na = jnp.sum(active, axis=0, keepdims=True)
    temax = jnp.max(te * active, axis=0, keepdims=True)
    te_fix = jnp.where(tcv > 0.0, te, temax)
    gidx = lax.broadcasted_iota(jnp.int32, (NW, 1), 0).astype(jnp.float32)
    xblk = jnp.minimum(gidx, na - 1.0)
    te_ref[...] = te_fix.reshape(1, NW).astype(jnp.int32)
    tc_ref[...] = tcv.reshape(1, NW).astype(jnp.int32)
    xb_ref[...] = xblk.reshape(1, NW).astype(jnp.int32)


def _route(x, wg):
    return pl.pallas_call(
        _router_body,
        out_shape=(
            jax.ShapeDtypeStruct((A, 1), jnp.int32),
            jax.ShapeDtypeStruct((1, NW), jnp.int32),
            jax.ShapeDtypeStruct((1, NW), jnp.int32),
            jax.ShapeDtypeStruct((1, NW), jnp.int32),
            jax.ShapeDtypeStruct((A, 16), jnp.float32),
        ),
    )(x, wg)


_NB = 4


_NBD = 6


def _dispatch_body(pos_hbm, x_hbm, xs_hbm, posb_v, *rest):
    pr = list(rest[:_NBD])
    br = list(rest[_NBD:2 * _NBD])
    sg = list(rest[2 * _NBD:3 * _NBD])
    ss = list(rest[3 * _NBD:4 * _NBD])
    wid = lax.axis_index("s") * 2 + lax.axis_index("c")
    base_a = wid * APT
    pltpu.sync_copy(pos_hbm.at[pl.ds(base_a, APT)], posb_v)
    iota = lax.iota(jnp.int32, 16)
    nv = APT // 16

    def gath(v):
        tvec = (jnp.full((16,), base_a + v * 16, jnp.int32) + iota) & (T - 1)
        return pltpu.async_copy(x_hbm.at[tvec], br[v % _NBD], sg[v % _NBD])

    gd = {v: gath(v) for v in range(_NBD)}
    sd = {}
    for v in range(nv):
        b = v % _NBD
        if v >= _NBD:
            sd[v - _NBD].wait()
            gd[v] = gath(v)
        gd[v].wait()
        pr[b][...] = posb_v[pl.ds(v * 16, 16)]
        sd[v] = pltpu.async_copy(br[b], xs_hbm.at[pr[b]], ss[b])
    for v in range(nv - _NBD, nv):
        sd[v].wait()


def _dispatch(pos_flat, x):
    mesh = plsc.VectorSubcoreMesh(core_axis_name="c", subcore_axis_name="s")
    return pl.kernel(
        _dispatch_body,
        out_type=jax.ShapeDtypeStruct((P_MAX, H), jnp.float32),
        mesh=mesh,
        scratch_types=(
            [pltpu.VMEM((APT,), jnp.int32)]
            + [pltpu.VMEM((16,), jnp.int32)] * _NBD
            + [pltpu.VMEM((16, H), jnp.float32)] * _NBD
            + [pltpu.SemaphoreType.DMA] * (2 * _NBD)
        ),
    )(pos_flat, x)


def _gmm_body(te_ref, tc_ref, xb_ref, x_ref, wgu_ref, wd_ref, y_ref):
    g = pl.program_id(0)
    cnt = tc_ref[g]

    @pl.when(cnt > 0)
    def _():
        x = x_ref[...].astype(jnp.bfloat16)
        h = jnp.dot(x, wgu_ref[0].astype(jnp.bfloat16),
                    preferred_element_type=jnp.float32)
        gate = h[:, :F]
        up = h[:, F:]
        act = gate * jax.nn.sigmoid(gate) * up
        y_ref[...] = jnp.dot(act.astype(jnp.bfloat16),
                             wd_ref[0].astype(jnp.bfloat16),
                             preferred_element_type=jnp.float32)


def _gmm(te, tcnt, xblk, x_sorted, w_gate_up, w_down):
    grid_spec = pltpu.PrefetchScalarGridSpec(
        num_scalar_prefetch=3,
        grid=(G,),
        in_specs=[
            pl.BlockSpec((TM, H), lambda g, te, tc, xb: (xb[g], 0)),
            pl.BlockSpec((1, H, 2 * F), lambda g, te, tc, xb: (te[g], 0, 0)),
            pl.BlockSpec((1, F, H), lambda g, te, tc, xb: (te[g], 0, 0)),
        ],
        out_specs=pl.BlockSpec((TM, H), lambda g, te, tc, xb: (xb[g], 0)),
    )
    return pl.pallas_call(
        _gmm_body,
        grid_spec=grid_spec,
        out_shape=jax.ShapeDtypeStruct((P_MAX, H), jnp.float32),
    )(te, tcnt, xblk, x_sorted, w_gate_up, w_down)


TPT = T // NW


def _combine_body(y_hbm, pos_hbm, wrep_hbm, out_hbm,
                  pb0_v, pb1_v, wb0_v, wb1_v,
                  ya0, yb0, ya1, yb1, ob0, ob1,
                  sa0, sb0, sa1, sb1, so0, so1):
    wid = lax.axis_index("s") * 2 + lax.axis_index("c")
    base_t = wid * TPT
    pltpu.sync_copy(pos_hbm.at[pl.ds(base_t, TPT)], pb0_v)
    pltpu.sync_copy(pos_hbm.at[pl.ds(T + base_t, TPT)], pb1_v)
    pltpu.sync_copy(wrep_hbm.at[pl.ds(base_t, TPT)], wb0_v)
    pltpu.sync_copy(wrep_hbm.at[pl.ds(T + base_t, TPT)], wb1_v)
    ya = [ya0, ya1]
    yb = [yb0, yb1]
    ob = [ob0, ob1]
    sa = [sa0, sa1]
    sb = [sb0, sb1]
    so = [so0, so1]
    nq = TPT // 16

    def gath(q):
        s = q % 2
        p0 = pb0_v[pl.ds(q * 16, 16)]
        p1 = pb1_v[pl.ds(q * 16, 16)]
        return (pltpu.async_copy(y_hbm.at[p0], ya[s], sa[s]),
                pltpu.async_copy(y_hbm.at[p1], yb[s], sb[s]))

    gd = {0: gath(0)}
    sd = {}
    for q in range(nq):
        s = q % 2
        gd[q][0].wait()
        gd[q][1].wait()
        if q + 1 < nq:
            gd[q + 1] = gath(q + 1)
        if q >= 2:
            sd[q - 2].wait()

        def row_body(i, carry, _q=q, _s=s):
            w0 = wb0_v[_q * 16 + i]
            w1 = wb1_v[_q * 16 + i]
            for c in range(H // 16):
                off = c * 16
                ob[_s][i, pl.ds(off, 16)] = (
                    w0 * ya[_s][i, pl.ds(off, 16)]
                    + w1 * yb[_s][i, pl.ds(off, 16)])
            return carry

        lax.fori_loop(0, 16, row_body, 0)
        sd[q] = pltpu.async_copy(
            ob[s], out_hbm.at[pl.ds(base_t + q * 16, 16)], so[s])
    sd[nq - 2].wait()
    sd[nq - 1].wait()


def _combine(y_sorted, pos_flat, wrep):
    mesh = plsc.VectorSubcoreMesh(core_axis_name="c", subcore_axis_name="s")
    return pl.kernel(
        _combine_body,
        out_type=jax.ShapeDtypeStruct((T, H), jnp.float32),
        mesh=mesh,
        scratch_types=(
            [pltpu.VMEM((TPT,), jnp.int32)] * 2
            + [pltpu.VMEM((TPT, 16), jnp.float32)] * 2
            + [pltpu.VMEM((16, H), jnp.float32)] * 6
            + [pltpu.SemaphoreType.DMA] * 6
        ),
    )(y_sorted, pos_flat, wrep)


def kernel(hidden_states, Wg, W_gate_up, W_down):
    x = hidden_states
    pos2d, te2d, tc2d, xb2d, wrep = _route(x, Wg)
    pos_flat = pos2d.reshape(A)
    x_sorted = _dispatch(pos_flat, x)
    te = te2d.reshape(NW)[:G]
    tcnt = tc2d.reshape(NW)[:G]
    xblk = xb2d.reshape(NW)[:G]
    y_sorted = _gmm(te, tcnt, xblk, x_sorted, W_gate_up, W_down)
    return _combine(y_sorted, pos_flat, wrep)


# speedup vs baseline: 5.0835x; 1.0944x over previous
import jax
import jax.numpy as jnp
from jax import lax
from jax.experimental import pallas as pl
from jax.experimental.pallas import tpu as pltpu
from jax.experimental.pallas import tpu_sc as plsc

E = 8
K = 2
H = 1024
F = 768
T = 2048
A = T * K
TM = 512
G = 16
P_MAX = G * TM

NW = 32
APT = A // NW
TMC = 256


def _router_body(x_ref, wg_ref, pos_ref, te_ref, tc_ref, xb_ref, wrep_ref):
    x = x_ref[...]
    wg = wg_ref[...]
    logits = lax.dot_general(x, wg, (((1,), (1,)), ((), ())),
                             preferred_element_type=jnp.float32)
    m = jnp.max(logits, axis=-1, keepdims=True)
    ex = jnp.exp(logits - m)
    probs = ex / jnp.sum(ex, axis=-1, keepdims=True)
    lane = lax.broadcasted_iota(jnp.int32, probs.shape, 1)
    v1 = jnp.max(probs, axis=-1, keepdims=True)
    i1 = jnp.argmax(probs, axis=-1).astype(jnp.int32)[:, None]
    masked = jnp.where(lane == i1, -jnp.inf, probs)
    v2 = jnp.max(masked, axis=-1, keepdims=True)
    i2 = jnp.argmax(masked, axis=-1).astype(jnp.int32)[:, None]
    s = v1 + v2
    wcol = jnp.concatenate([v1 / s, v2 / s], axis=0)
    wrep_ref[...] = jnp.broadcast_to(wcol, (A, 16))

    oh = jnp.concatenate([(lane == i1).astype(jnp.float32),
                          (lane == i2).astype(jnp.float32)], axis=0)
    cs = oh
    k = 1
    while k < A:
        cs = cs + jnp.concatenate(
            [jnp.zeros((k, E), jnp.float32), cs[:A - k]], axis=0)
        k *= 2
    counts = cs[A - 1:A]
    padded = jnp.ceil(counts / TM) * TM
    lower8 = (lax.broadcasted_iota(jnp.int32, (E, E), 0)
              <= lax.broadcasted_iota(jnp.int32, (E, E), 1)).astype(jnp.float32)
    incl = jnp.dot(padded, lower8, preferred_element_type=jnp.float32)
    poff = incl - padded
    pos = jnp.sum(oh * (cs - 1.0 + poff), axis=1, keepdims=True)
    pos_ref[...] = pos.astype(jnp.int32)

    ts = lax.broadcasted_iota(jnp.int32, (NW, E), 0).astype(jnp.float32) * TM
    acc = jnp.sum((incl <= ts).astype(jnp.float32), axis=1, keepdims=True)
    te = jnp.clip(acc, 0.0, E - 1)
    teoh = (lax.broadcasted_iota(jnp.int32, (NW, E), 1).astype(jnp.float32)
            == te).astype(jnp.float32)
    cnt_te = jnp.sum(teoh * counts, axis=1, keepdims=True)
    poff_te = jnp.sum(teoh * poff, axis=1, keepdims=True)
    tcv = jnp.clip(cnt_te - (ts[:, :1] - poff_te), 0.0, TM)
    active = (tcv > 0.0).astype(jnp.float32)
    na = jnp.sum(active, axis=0, keepdims=True)
    temax = jnp.max(te * active, axis=0, keepdims=True)
    te_fix = jnp.where(tcv > 0.0, te, temax)
    gidx = lax.broadcasted_iota(jnp.int32, (NW, 1), 0).astype(jnp.float32)
    xblk = jnp.minimum(gidx, na - 1.0)
    te_ref[...] = te_fix.reshape(1, NW).astype(jnp.int32)
    tc_ref[...] = tcv.reshape(1, NW).astype(jnp.int32)
    xb_ref[...] = xblk.reshape(1, NW).astype(jnp.int32)


def _route(x, wg):
    return pl.pallas_call(
        _router_body,
        out_shape=(
            jax.ShapeDtypeStruct((A, 1), jnp.int32),
            jax.ShapeDtypeStruct((1, NW), jnp.int32),
            jax.ShapeDtypeStruct((1, NW), jnp.int32),
            jax.ShapeDtypeStruct((1, NW), jnp.int32),
            jax.ShapeDtypeStruct((A, 16), jnp.float32),
        ),
    )(x, wg)


_NB = 4


_NBD = 6


def _dispatch_body(pos_hbm, x_hbm, xs_hbm, posb_v, *rest):
    pr = list(rest[:_NBD])
    br = list(rest[_NBD:2 * _NBD])
    sg = list(rest[2 * _NBD:3 * _NBD])
    ss = list(rest[3 * _NBD:4 * _NBD])
    wid = lax.axis_index("s") * 2 + lax.axis_index("c")
    base_a = wid * APT
    pltpu.sync_copy(pos_hbm.at[pl.ds(base_a, APT)], posb_v)
    iota = lax.iota(jnp.int32, 16)
    nv = APT // 16

    def gath(v):
        tvec = (jnp.full((16,), base_a + v * 16, jnp.int32) + iota) & (T - 1)
        return pltpu.async_copy(x_hbm.at[tvec], br[v % _NBD], sg[v % _NBD])

    gd = {v: gath(v) for v in range(_NBD)}
    sd = {}
    for v in range(nv):
        b = v % _NBD
        if v >= _NBD:
            sd[v - _NBD].wait()
            gd[v] = gath(v)
        gd[v].wait()
        pr[b][...] = posb_v[pl.ds(v * 16, 16)]
        sd[v] = pltpu.async_copy(br[b], xs_hbm.at[pr[b]], ss[b])
    for v in range(nv - _NBD, nv):
        sd[v].wait()


def _dispatch(pos_flat, x):
    mesh = plsc.VectorSubcoreMesh(core_axis_name="c", subcore_axis_name="s")
    return pl.kernel(
        _dispatch_body,
        out_type=jax.ShapeDtypeStruct((P_MAX, H), jnp.float32),
        mesh=mesh,
        scratch_types=(
            [pltpu.VMEM((APT,), jnp.int32)]
            + [pltpu.VMEM((16,), jnp.int32)] * _NBD
            + [pltpu.VMEM((16, H), jnp.float32)] * _NBD
            + [pltpu.SemaphoreType.DMA] * (2 * _NBD)
        ),
    )(pos_flat, x)


def _gmm_body(te_ref, tc_ref, xb_ref, x_ref, wgu_ref, wd_ref, y_ref):
    g = pl.program_id(0)
    cnt = tc_ref[g]

    @pl.when(cnt > 0)
    def _():
        x = x_ref[...].astype(jnp.bfloat16)
        h = jnp.dot(x, wgu_ref[0].astype(jnp.bfloat16),
                    preferred_element_type=jnp.float32)
        gate = h[:, :F]
        up = h[:, F:]
        act = gate * jax.nn.sigmoid(gate) * up
        y_ref[...] = jnp.dot(act.astype(jnp.bfloat16),
                             wd_ref[0].astype(jnp.bfloat16),
                             preferred_element_type=jnp.float32)


def _gmm(te, tcnt, xblk, x_sorted, w_gate_up, w_down):
    grid_spec = pltpu.PrefetchScalarGridSpec(
        num_scalar_prefetch=3,
        grid=(G,),
        in_specs=[
            pl.BlockSpec((TM, H), lambda g, te, tc, xb: (xb[g], 0)),
            pl.BlockSpec((1, H, 2 * F), lambda g, te, tc, xb: (te[g], 0, 0)),
            pl.BlockSpec((1, F, H), lambda g, te, tc, xb: (te[g], 0, 0)),
        ],
        out_specs=pl.BlockSpec((TM, H), lambda g, te, tc, xb: (xb[g], 0)),
    )
    return pl.pallas_call(
        _gmm_body,
        grid_spec=grid_spec,
        out_shape=jax.ShapeDtypeStruct((P_MAX, H), jnp.float32),
    )(te, tcnt, xblk, x_sorted, w_gate_up, w_down)


TPT = T // NW


def _combine_body(y_hbm, pos_hbm, wrep_hbm, out_hbm,
                  pb0_v, pb1_v, wb0_v, wb1_v,
                  ya0, yb0, ya1, yb1, ob0, ob1,
                  sa0, sb0, sa1, sb1, so0, so1):
    wid = lax.axis_index("s") * 2 + lax.axis_index("c")
    base_t = wid * TPT
    pltpu.sync_copy(pos_hbm.at[pl.ds(base_t, TPT)], pb0_v)
    pltpu.sync_copy(pos_hbm.at[pl.ds(T + base_t, TPT)], pb1_v)
    pltpu.sync_copy(wrep_hbm.at[pl.ds(base_t, TPT)], wb0_v)
    pltpu.sync_copy(wrep_hbm.at[pl.ds(T + base_t, TPT)], wb1_v)
    ya = [ya0, ya1]
    yb = [yb0, yb1]
    ob = [ob0, ob1]
    sa = [sa0, sa1]
    sb = [sb0, sb1]
    so = [so0, so1]
    nq = TPT // 16

    def gath(q):
        s = q % 2
        p0 = pb0_v[pl.ds(q * 16, 16)]
        p1 = pb1_v[pl.ds(q * 16, 16)]
        return (pltpu.async_copy(y_hbm.at[p0], ya[s], sa[s]),
                pltpu.async_copy(y_hbm.at[p1], yb[s], sb[s]))

    gd = {0: gath(0)}
    sd = {}
    for q in range(nq):
        s = q % 2
        gd[q][0].wait()
        gd[q][1].wait()
        if q + 1 < nq:
            gd[q + 1] = gath(q + 1)
        if q >= 2:
            sd[q - 2].wait()

        def row_body(i, carry, _q=q, _s=s):
            w0 = wb0_v[_q * 16 + i]
            w1 = wb1_v[_q * 16 + i]
            for c in range(H // 16):
                off = c * 16
                ob[_s][i, pl.ds(off, 16)] = (
                    w0 * ya[_s][i, pl.ds(off, 16)]
                    + w1 * yb[_s][i, pl.ds(off, 16)])
            return carry

        lax.fori_loop(0, 16, row_body, 0)
        sd[q] = pltpu.async_copy(
            ob[s], out_hbm.at[pl.ds(base_t + q * 16, 16)], so[s])
    sd[nq - 2].wait()
    sd[nq - 1].wait()


def _combine(y_sorted, pos_flat, wrep):
    mesh = plsc.VectorSubcoreMesh(core_axis_name="c", subcore_axis_name="s")
    return pl.kernel(
        _combine_body,
        out_type=jax.ShapeDtypeStruct((T, H), jnp.float32),
        mesh=mesh,
        scratch_types=(
            [pltpu.VMEM((TPT,), jnp.int32)] * 2
            + [pltpu.VMEM((TPT, 16), jnp.float32)] * 2
            + [pltpu.VMEM((16, H), jnp.float32)] * 6
            + [pltpu.SemaphoreType.DMA] * 6
        ),
    )(y_sorted, pos_flat, wrep)


def kernel(hidden_states, Wg, W_gate_up, W_down):
    x = hidden_states
    pos2d, te2d, tc2d, xb2d, wrep = _route(x, Wg)
    pos_flat = pos2d.reshape(A)
    x_sorted = _dispatch(pos_flat, x)
    te = te2d.reshape(NW)[:G]
    tcnt = tc2d.reshape(NW)[:G]
    xblk = xb2d.reshape(NW)[:G]
    y_sorted = _gmm(te, tcnt, xblk, x_sorted, W_gate_up, W_down)
    return _combine(y_sorted, pos_flat, wrep)
